# Initial kernel scaffold; baseline (speedup 1.0000x reference)
import jax
import jax.numpy as jnp
from jax.experimental import pallas as pl


def kernel(x, weight, bias):
    raise NotImplementedError("write your pallas kernel here")


# speedup vs baseline: 10.0216x; 10.0216x over previous
import jax
import jax.numpy as jnp
from jax.experimental import pallas as pl

_TOPK = 64
_BR = 256
_POS_INF_BITS = 0x7F800000


def _body(x_ref, w_ref, b_ref, o_ref):
    y = jax.lax.dot_general(
        x_ref[...], w_ref[...],
        dimension_numbers=(((1,), (1,)), ((), ())),
        preferred_element_type=jnp.float32,
    ) + b_ref[...]
    a = jax.lax.bitcast_convert_type(jnp.abs(y), jnp.int32)

    def step(_, carry):
        lo, hi = carry
        mid = lo + ((hi - lo) >> 1)
        cnt = jnp.sum((a >= mid).astype(jnp.int32), axis=1, keepdims=True)
        ge = cnt >= _TOPK
        return jnp.where(ge, mid, lo), jnp.where(ge, hi, mid)

    rows = a.shape[0]
    lo0 = jnp.zeros((rows, 1), jnp.int32)
    hi0 = jnp.full((rows, 1), _POS_INF_BITS, jnp.int32)
    lo, _ = jax.lax.fori_loop(0, 31, step, (lo0, hi0))
    o_ref[...] = jnp.where(a >= lo, y, 0.0)


def kernel(x, weight, bias):
    n, fin = x.shape
    fout = weight.shape[0]
    return pl.pallas_call(
        _body,
        grid=(n // _BR,),
        in_specs=[
            pl.BlockSpec((_BR, fin), lambda i: (i, 0)),
            pl.BlockSpec((fout, fin), lambda i: (0, 0)),
            pl.BlockSpec((1, fout), lambda i: (0, 0)),
        ],
        out_specs=pl.BlockSpec((_BR, fout), lambda i: (i, 0)),
        out_shape=jax.ShapeDtypeStruct((n, fout), jnp.float32),
    )(x, weight, bias.reshape(1, fout))
